# Initial kernel scaffold; baseline (speedup 1.0000x reference)
import jax
import jax.numpy as jnp
from jax.experimental import pallas as pl


def kernel(x_in, in_len, table, Wih_f, Whh_f, bih_f, bhh_f, Wih_b, Whh_b, bih_b, bhh_b, Wl, bl):
    raise NotImplementedError("write your pallas kernel here")


# speedup vs baseline: 4.1785x; 4.1785x over previous
import functools

import jax
import jax.numpy as jnp
from jax import lax
from jax.experimental import pallas as pl
from jax.experimental.pallas import tpu as pltpu
from jax.experimental.pallas import tpu_sc as plsc

D = 128
H = 128
B = 1024
L = 50

NW = 32
ROWS_W = B * L // NW
CH = 80
NCH = ROWS_W // CH


def _sc_gather(idx, table):
    mesh = plsc.VectorSubcoreMesh(core_axis_name="c", subcore_axis_name="s")

    @functools.partial(
        pl.kernel,
        mesh=mesh,
        out_type=jax.ShapeDtypeStruct((B * L, D), jnp.float32),
        scratch_types=[
            pltpu.VMEM((NCH, CH), jnp.int32),
            pltpu.VMEM((CH, D), jnp.float32),
            pltpu.SemaphoreType.DMA,
        ],
    )
    def k(idx_hbm, table_hbm, out_hbm, idx_v, rows_v, sem):
        wid = lax.axis_index("s") * 2 + lax.axis_index("c")
        pltpu.sync_copy(idx_hbm.at[wid], idx_v)
        base = wid * ROWS_W

        def chunk(j, carry):
            pltpu.async_copy(table_hbm.at[idx_v.at[j]], rows_v, sem).wait()
            pltpu.sync_copy(rows_v, out_hbm.at[pl.ds(base + j * CH, CH)])
            return carry

        lax.fori_loop(0, NCH, chunk, 0)

    return k(idx, table)


def _gru_body(emb_ref, lens_ref, wihf_ref, whhf_ref, bihf_ref, bhhf_ref,
              wihb_ref, whhb_ref, bihb_ref, bhhb_ref, wlf_ref, wlb_ref,
              bl_ref, out_ref, hf_ref):
    wihf = wihf_ref[...]
    whhf = whhf_ref[...]
    bihf = bihf_ref[...]
    bhhf = bhhf_ref[...]
    wihb = wihb_ref[...]
    whhb = whhb_ref[...]
    bihb = bihb_ref[...]
    bhhb = bhhb_ref[...]
    wlf = wlf_ref[...]
    wlb = wlb_ref[...]
    bl = bl_ref[...]
    lens = lens_ref[...]
    linv = 1.0 / lens.astype(jnp.float32)

    def gru_step(x, h, wih, whh, bih, bhh):
        gi = jnp.dot(x, wih, preferred_element_type=jnp.float32) + bih
        gh = jnp.dot(h, whh, preferred_element_type=jnp.float32) + bhh
        r = jax.nn.sigmoid(gi[:, :H] + gh[:, :H])
        z = jax.nn.sigmoid(gi[:, H:2 * H] + gh[:, H:2 * H])
        n = jnp.tanh(gi[:, 2 * H:] + r * gh[:, 2 * H:])
        return (1.0 - z) * n + z * h

    h0 = jnp.zeros((B, H), jnp.float32)

    def fwd(t, h):
        h = gru_step(emb_ref[t], h, wihf, whhf, bihf, bhhf)
        hf_ref[t] = h
        return h

    lax.fori_loop(0, L, fwd, h0)

    def bwd(i, carry):
        h, acc = carry
        t = L - 1 - i
        h = gru_step(emb_ref[t], h, wihb, whhb, bihb, bhhb)
        y = jnp.dot(hf_ref[t], wlf, preferred_element_type=jnp.float32)
        y = y + jnp.dot(h, wlb, preferred_element_type=jnp.float32) + bl
        y = jnp.maximum(y, 0.0)
        wc = jnp.sum(jnp.where(lens > t, linv, 0.0))
        acc = acc + wc * jnp.sum(y, axis=0, keepdims=True)
        return h, acc

    _, acc = lax.fori_loop(0, L, bwd, (h0, jnp.zeros((1, D), jnp.float32)))
    out_ref[...] = acc


def _tc_gru(emb_lbd, lens_i, wihf, whhf, bihf, bhhf, wihb, whhb, bihb, bhhb,
            wlf, wlb, bl2):
    return pl.pallas_call(
        _gru_body,
        out_shape=jax.ShapeDtypeStruct((1, D), jnp.float32),
        scratch_shapes=[pltpu.VMEM((L, B, H), jnp.float32)],
    )(emb_lbd, lens_i, wihf, whhf, bihf, bhhf, wihb, whhb, bihb, bhhb,
      wlf, wlb, bl2)


def kernel(x_in, in_len, table, Wih_f, Whh_f, bih_f, bhh_f,
           Wih_b, Whh_b, bih_b, bhh_b, Wl, bl):
    x_in = x_in.astype(jnp.int32)
    idx = x_in.T.reshape(NW, NCH, CH)
    emb = _sc_gather(idx, table).reshape(L, B, D)

    lens_i = in_len.astype(jnp.int32).reshape(8, 128)
    out = _tc_gru(
        emb, lens_i,
        Wih_f.T, Whh_f.T, bih_f.reshape(1, 3 * H), bhh_f.reshape(1, 3 * H),
        Wih_b.T, Whh_b.T, bih_b.reshape(1, 3 * H), bhh_b.reshape(1, 3 * H),
        Wl[:, :H].T, Wl[:, H:].T, bl.reshape(1, D),
    )
    return out


# speedup vs baseline: 5.0695x; 1.2132x over previous
import functools

import jax
import jax.numpy as jnp
from jax import lax
from jax.experimental import pallas as pl
from jax.experimental.pallas import tpu as pltpu
from jax.experimental.pallas import tpu_sc as plsc

D = 128
H = 128
B = 1024
L = 50

NW = 32
ROWS_W = B * L // NW
CH = 80
NCH = ROWS_W // CH

bf16 = jnp.bfloat16


def _sc_gather(idx, table):
    mesh = plsc.VectorSubcoreMesh(core_axis_name="c", subcore_axis_name="s")

    @functools.partial(
        pl.kernel,
        mesh=mesh,
        out_type=jax.ShapeDtypeStruct((B * L, D), jnp.float32),
        scratch_types=[
            pltpu.VMEM((NCH, CH), jnp.int32),
            pltpu.VMEM((2, CH, D), jnp.float32),
            pltpu.SemaphoreType.DMA,
            pltpu.SemaphoreType.DMA,
        ],
    )
    def k(idx_hbm, table_hbm, out_hbm, idx_v, rows_v, sem0, sem1):
        wid = lax.axis_index("s") * 2 + lax.axis_index("c")
        pltpu.sync_copy(idx_hbm.at[wid], idx_v)
        base = wid * ROWS_W
        pltpu.async_copy(table_hbm.at[idx_v.at[0]], rows_v.at[0], sem0)
        pltpu.async_copy(table_hbm.at[idx_v.at[1]], rows_v.at[1], sem1)

        def pair(jj, carry):
            j0 = 2 * jj
            pltpu.make_async_copy(
                table_hbm.at[idx_v.at[j0]], rows_v.at[0], sem0).wait()
            pltpu.sync_copy(rows_v.at[0],
                            out_hbm.at[pl.ds(base + j0 * CH, CH)])

            @pl.when(jj < NCH // 2 - 1)
            def _():
                pltpu.async_copy(
                    table_hbm.at[idx_v.at[j0 + 2]], rows_v.at[0], sem0)

            pltpu.make_async_copy(
                table_hbm.at[idx_v.at[j0 + 1]], rows_v.at[1], sem1).wait()
            pltpu.sync_copy(rows_v.at[1],
                            out_hbm.at[pl.ds(base + (j0 + 1) * CH, CH)])

            @pl.when(jj < NCH // 2 - 1)
            def _():
                pltpu.async_copy(
                    table_hbm.at[idx_v.at[j0 + 3]], rows_v.at[1], sem1)

            return carry

        lax.fori_loop(0, NCH // 2, pair, 0)

    return k(idx, table)


def _gru_body(emb_ref, lens_ref, wihf_ref, whhf_ref, bihf_ref, bhhf_ref,
              wihb_ref, whhb_ref, bihb_ref, bhhb_ref, wlf_ref, wlb_ref,
              bl_ref, out_ref, hf_ref, hb_ref):
    wihf = wihf_ref[...]
    whhf = whhf_ref[...]
    bihf = bihf_ref[...]
    bhhf = bhhf_ref[...]
    wihb = wihb_ref[...]
    whhb = whhb_ref[...]
    bihb = bihb_ref[...]
    bhhb = bhhb_ref[...]
    wlf = wlf_ref[...]
    wlb = wlb_ref[...]
    bl = bl_ref[...]
    lens = lens_ref[...]
    linv = 1.0 / lens.astype(jnp.float32)

    def cell(x, h, wih, whh, bih, bhh):
        gi = jnp.dot(x.astype(bf16), wih,
                     preferred_element_type=jnp.float32) + bih
        gh = jnp.dot(h.astype(bf16), whh,
                     preferred_element_type=jnp.float32) + bhh
        r = jax.nn.sigmoid(gi[:, :H] + gh[:, :H])
        z = jax.nn.sigmoid(gi[:, H:2 * H] + gh[:, H:2 * H])
        n = jnp.tanh(gi[:, 2 * H:] + r * gh[:, 2 * H:])
        return n + z * (h - n)

    h0 = jnp.zeros((B, H), jnp.float32)

    def step(i, carry):
        hf, hb = carry
        tb = L - 1 - i
        hf = cell(emb_ref[i], hf, wihf, whhf, bihf, bhhf)
        hb = cell(emb_ref[tb], hb, wihb, whhb, bihb, bhhb)
        hf_ref[i] = hf.astype(bf16)
        hb_ref[tb] = hb.astype(bf16)
        return hf, hb

    lax.fori_loop(0, L, step, (h0, h0))

    def comb(t, acc):
        y = jnp.dot(hf_ref[t], wlf, preferred_element_type=jnp.float32)
        y = y + jnp.dot(hb_ref[t], wlb, preferred_element_type=jnp.float32)
        y = jnp.maximum(y + bl, 0.0)
        wc = jnp.sum(jnp.where(lens > t, linv, 0.0))
        return acc + wc * jnp.sum(y, axis=0, keepdims=True)

    out_ref[...] = lax.fori_loop(0, L, comb, jnp.zeros((1, D), jnp.float32))


def _tc_gru(emb_lbd, lens_i, wihf, whhf, bihf, bhhf, wihb, whhb, bihb, bhhb,
            wlf, wlb, bl2):
    return pl.pallas_call(
        _gru_body,
        out_shape=jax.ShapeDtypeStruct((1, D), jnp.float32),
        scratch_shapes=[pltpu.VMEM((L, B, H), bf16),
                        pltpu.VMEM((L, B, H), bf16)],
    )(emb_lbd, lens_i, wihf, whhf, bihf, bhhf, wihb, whhb, bihb, bhhb,
      wlf, wlb, bl2)


def kernel(x_in, in_len, table, Wih_f, Whh_f, bih_f, bhh_f,
           Wih_b, Whh_b, bih_b, bhh_b, Wl, bl):
    x_in = x_in.astype(jnp.int32)
    idx = x_in.T.reshape(NW, NCH, CH)
    emb = _sc_gather(idx, table).reshape(L, B, D)

    lens_i = in_len.astype(jnp.int32).reshape(8, 128)
    out = _tc_gru(
        emb, lens_i,
        Wih_f.T.astype(bf16), Whh_f.T.astype(bf16),
        bih_f.reshape(1, 3 * H), bhh_f.reshape(1, 3 * H),
        Wih_b.T.astype(bf16), Whh_b.T.astype(bf16),
        bih_b.reshape(1, 3 * H), bhh_b.reshape(1, 3 * H),
        Wl[:, :H].T.astype(bf16), Wl[:, H:].T.astype(bf16),
        bl.reshape(1, D),
    )
    return out


# speedup vs baseline: 5.9399x; 1.1717x over previous
import functools

import jax
import jax.numpy as jnp
from jax import lax
from jax.experimental import pallas as pl
from jax.experimental.pallas import tpu as pltpu
from jax.experimental.pallas import tpu_sc as plsc

D = 128
H = 128
B = 1024
L = 50

NW = 32
ROWS_W = B * L // NW
CH = 80
NCH = ROWS_W // CH

bf16 = jnp.bfloat16


def _sc_gather(idx, table):
    mesh = plsc.VectorSubcoreMesh(core_axis_name="c", subcore_axis_name="s")

    @functools.partial(
        pl.kernel,
        mesh=mesh,
        out_type=jax.ShapeDtypeStruct((B * L, D), jnp.float32),
        scratch_types=[
            pltpu.VMEM((NCH, CH), jnp.int32),
            pltpu.VMEM((2, CH, D), jnp.float32),
            pltpu.SemaphoreType.DMA,
            pltpu.SemaphoreType.DMA,
        ],
    )
    def k(idx_hbm, table_hbm, out_hbm, idx_v, rows_v, sem0, sem1):
        wid = lax.axis_index("s") * 2 + lax.axis_index("c")
        pltpu.sync_copy(idx_hbm.at[wid], idx_v)
        base = wid * ROWS_W
        pltpu.async_copy(table_hbm.at[idx_v.at[0]], rows_v.at[0], sem0)
        pltpu.async_copy(table_hbm.at[idx_v.at[1]], rows_v.at[1], sem1)

        def pair(jj, carry):
            j0 = 2 * jj
            pltpu.make_async_copy(
                table_hbm.at[idx_v.at[j0]], rows_v.at[0], sem0).wait()
            pltpu.sync_copy(rows_v.at[0],
                            out_hbm.at[pl.ds(base + j0 * CH, CH)])

            @pl.when(jj < NCH // 2 - 1)
            def _():
                pltpu.async_copy(
                    table_hbm.at[idx_v.at[j0 + 2]], rows_v.at[0], sem0)

            pltpu.make_async_copy(
                table_hbm.at[idx_v.at[j0 + 1]], rows_v.at[1], sem1).wait()
            pltpu.sync_copy(rows_v.at[1],
                            out_hbm.at[pl.ds(base + (j0 + 1) * CH, CH)])

            @pl.when(jj < NCH // 2 - 1)
            def _():
                pltpu.async_copy(
                    table_hbm.at[idx_v.at[j0 + 3]], rows_v.at[1], sem1)

            return carry

        lax.fori_loop(0, NCH // 2, pair, 0)

    return k(idx, table)


def _gru_body(emb_ref, lens_ref, wrzf_ref, winf_ref, whnf_ref, brzf_ref,
              binf_ref, bhnf_ref, wrzb_ref, winb_ref, whnb_ref, brzb_ref,
              binb_ref, bhnb_ref, wcomb_ref, bl_ref, out_ref, hfb_ref):
    lens = lens_ref[...]
    linv = 1.0 / lens.astype(jnp.float32)

    def sig(x):
        return 0.5 * jnp.tanh(0.5 * x) + 0.5

    def cell(x, h, wrz_ref, win_ref, whn_ref, brz_ref, bin_ref, bhn_ref):
        xb = x.astype(bf16)
        hb = h.astype(bf16)
        xh = jnp.concatenate([xb, hb], axis=1)
        a_rz = jnp.dot(xh, wrz_ref[...],
                       preferred_element_type=jnp.float32) + brz_ref[...]
        hn = jnp.dot(hb, whn_ref[...],
                     preferred_element_type=jnp.float32) + bhn_ref[...]
        i_n = jnp.dot(xb, win_ref[...],
                      preferred_element_type=jnp.float32) + bin_ref[...]
        r = sig(a_rz[:, :H])
        z = sig(a_rz[:, H:])
        n = jnp.tanh(i_n + r * hn)
        return n + z * (h - n)

    h0 = jnp.zeros((B, H), jnp.float32)

    def step(i, carry):
        hf, hb = carry
        tb = L - 1 - i
        hf = cell(emb_ref[i], hf, wrzf_ref, winf_ref, whnf_ref,
                  brzf_ref, binf_ref, bhnf_ref)
        hb = cell(emb_ref[tb], hb, wrzb_ref, winb_ref, whnb_ref,
                  brzb_ref, binb_ref, bhnb_ref)
        hfb_ref[i, :, :H] = hf.astype(bf16)
        hfb_ref[tb, :, H:] = hb.astype(bf16)
        return hf, hb

    lax.fori_loop(0, L, step, (h0, h0), unroll=2)

    def comb(t, acc):
        y = jnp.dot(hfb_ref[t], wcomb_ref[...],
                    preferred_element_type=jnp.float32)
        y = jnp.maximum(y + bl_ref[...], 0.0)
        wc = jnp.sum(jnp.where(lens > t, linv, 0.0))
        return acc + wc * jnp.sum(y, axis=0, keepdims=True)

    out_ref[...] = lax.fori_loop(0, L, comb, jnp.zeros((1, D), jnp.float32),
                                 unroll=2)


def _tc_gru(emb_lbd, lens_i, *weights):
    return pl.pallas_call(
        _gru_body,
        out_shape=jax.ShapeDtypeStruct((1, D), jnp.float32),
        scratch_shapes=[pltpu.VMEM((L, B, 2 * H), bf16)],
    )(emb_lbd, lens_i, *weights)


def _dir_weights(Wih, Whh, bih, bhh):
    wihT = Wih.T.astype(bf16)
    whhT = Whh.T.astype(bf16)
    wrz = jnp.concatenate([wihT[:, :2 * H], whhT[:, :2 * H]], axis=0)
    win = wihT[:, 2 * H:]
    whn = whhT[:, 2 * H:]
    brz = (bih[:2 * H] + bhh[:2 * H]).reshape(1, 2 * H)
    bin_ = bih[2 * H:].reshape(1, H)
    bhn = bhh[2 * H:].reshape(1, H)
    return wrz, win, whn, brz, bin_, bhn


def kernel(x_in, in_len, table, Wih_f, Whh_f, bih_f, bhh_f,
           Wih_b, Whh_b, bih_b, bhh_b, Wl, bl):
    x_in = x_in.astype(jnp.int32)
    idx = x_in.T.reshape(NW, NCH, CH)
    emb = _sc_gather(idx, table).reshape(L, B, D)

    lens_i = in_len.astype(jnp.int32).reshape(8, 128)
    out = _tc_gru(
        emb, lens_i,
        *_dir_weights(Wih_f, Whh_f, bih_f, bhh_f),
        *_dir_weights(Wih_b, Whh_b, bih_b, bhh_b),
        Wl.T.astype(bf16), bl.reshape(1, D),
    )
    return out
